# Initial kernel scaffold; baseline (speedup 1.0000x reference)
import jax
import jax.numpy as jnp
from jax.experimental import pallas as pl


def kernel(x, context, permutation):
    raise NotImplementedError("write your pallas kernel here")


# speedup vs baseline: 4.3544x; 4.3544x over previous
import jax
import jax.numpy as jnp
from jax import lax
from jax.experimental import pallas as pl

_ROWS_PER_BLOCK = 4096


def _permute_body(perm_ref, x_ref, out_ref):
    pid = perm_ref[...]
    c = pid.shape[0]
    cols = lax.broadcasted_iota(jnp.int32, (c, c), 0)
    p = (cols == pid[None, :]).astype(jnp.float32)
    out_ref[...] = jnp.dot(x_ref[...], p, preferred_element_type=jnp.float32)


def kernel(x, context, permutation):
    n, c = x.shape
    out = pl.pallas_call(
        _permute_body,
        grid=(n // _ROWS_PER_BLOCK,),
        in_specs=[
            pl.BlockSpec((c,), lambda i: (0,)),
            pl.BlockSpec((_ROWS_PER_BLOCK, c), lambda i: (i, 0)),
        ],
        out_specs=pl.BlockSpec((_ROWS_PER_BLOCK, c), lambda i: (i, 0)),
        out_shape=jax.ShapeDtypeStruct((n, c), x.dtype),
    )(permutation, x)
    log_J = jnp.zeros((n, 1), dtype=x.dtype)
    return (out, log_J, 0.0)
